# baseline (speedup 1.0000x reference)
import jax
import jax.numpy as jnp
from jax.experimental import pallas as pl
from jax.experimental.pallas import tpu as pltpu

_DROP_LENGTH_LOW = 1000
_DROP_LENGTH_HIGH = 8000
_DROP_COUNT_LOW = 1
_DROP_COUNT_HIGH = 10
_SEED = 42

_B = 32
_T = 160000
_CB = 6400
_MAXD = _DROP_COUNT_HIGH


def _interval_params(lengths):
    T = _T
    key = jax.random.key(_SEED)
    kp, kc, kl, ks = jax.random.split(key, 4)
    clean_length = (lengths * T).astype(jnp.int32)
    drop_times = jax.random.randint(kc, (_B,), _DROP_COUNT_LOW, _DROP_COUNT_HIGH)
    chunk_len = jax.random.randint(
        kl, (_B, _MAXD), _DROP_LENGTH_LOW, _DROP_LENGTH_HIGH + 1)
    u = jax.random.uniform(ks, (_B, _MAXD))
    max_start = jnp.maximum(clean_length[:, None] - chunk_len, 1)
    start = (u * max_start.astype(jnp.float32)).astype(jnp.int32)
    valid = jnp.arange(_MAXD)[None, :] < drop_times[:, None]
    end = jnp.where(valid, start + chunk_len, start)
    return start, end


def _body(start_ref, end_ref, w_ref, o_ref):
    j = pl.program_id(0)
    pos = jax.lax.broadcasted_iota(jnp.int32, (_B, _CB), 1) + j * _CB
    keep = None
    for d in range(_MAXD):
        s = start_ref[:, d][:, None]
        e = end_ref[:, d][:, None]
        drop = (pos >= s) & (pos < e)
        keep = drop if keep is None else (keep | drop)
    o_ref[...] = jnp.where(keep, 0.0, w_ref[...])


def kernel(waveform, lengths):
    start, end = _interval_params(lengths)
    grid = _T // _CB
    return pl.pallas_call(
        _body,
        grid=(grid,),
        in_specs=[
            pl.BlockSpec((_B, _MAXD), lambda j: (0, 0)),
            pl.BlockSpec((_B, _MAXD), lambda j: (0, 0)),
            pl.BlockSpec((_B, _CB), lambda j: (0, j)),
        ],
        out_specs=pl.BlockSpec((_B, _CB), lambda j: (0, j)),
        out_shape=jax.ShapeDtypeStruct((_B, _T), jnp.float32),
    )(start, end, waveform)
